# baseline (speedup 1.0000x reference)
import numpy as np
import jax
import jax.numpy as jnp
from jax.experimental import pallas as pl
from jax.experimental.pallas import tpu as pltpu

B = 32
HIST = 8
PRED = 24
CITY = 10
NSTA = 12
STA = 120
AQI_EM = 32
RNN_H = 64
GNN_H = 32

_F32 = jnp.float32
_BF16 = jnp.bfloat16


def _dot(a, b):
    return jnp.dot(a, b, preferred_element_type=_F32)


def _block_mask(width):
    lane = jax.lax.broadcasted_iota(jnp.int32, (CITY, CITY * width), 1)
    sub = jax.lax.broadcasted_iota(jnp.int32, (CITY, CITY * width), 0)
    return jnp.where(lane // width == sub, 1.0, 0.0).astype(_F32)


def _tile_lanes(x, n):
    return jnp.concatenate([x] * n, axis=1)


def _to_row(per_city, mask):
    return jnp.sum(mask * _tile_lanes(per_city, CITY), axis=0, keepdims=True)


def _fused_body(
    cxh_ref,
    xst_ref,
    cmt_ref,
    cdd_ref,
    gbias_ref,
    gemW_ref,
    gWz_ref, gWr_ref, gWn_ref,
    gUz_ref, gUr_ref, gUn_ref,
    gmW1_ref,
    gmW2_ref,
    gdW_ref,
    cemW_ref,
    cemb_ref,
    ch0W_ref,
    cWz_ref, cWr_ref, cWn_ref,
    cUz_ref, cUr_ref, cUn_ref,
    cbz_ref, cbr_ref, cbn_ref,
    cmW1_ref,
    cmb1_ref,
    cmW2_ref,
    cmb2_ref,
    cdW_ref,
    cdb_ref,
    fcW_ref,
    cei_ref,
    ei_ref,
    cea_ref,
    sea_ref,
    ccf_ref,
    out_ref,
    hg_ref,
    gsrc_ref,
    gdst_ref,
    eawg_ref,
    aggg_ref,
    hs_ref,
    ssrc_ref,
    sdst_ref,
    eaws_ref,
    ags_ref,
    UzrBD_ref,
    UnBD_ref,
    W1aBD_ref,
    W1bBD_ref,
    W2BD_ref,
    D1BD_ref,
    D2BD_ref,
):
    H = RNN_H
    G = GNN_H
    CH = CITY * H
    CG = CITY * G
    CP = CITY * PRED
    NB = NSTA * B

    gb = gbias_ref[...]
    gemb = gb[:, 0:32]
    gbz = gb[:, 32:96]
    gbr = gb[:, 96:160]
    gbn = gb[:, 160:224]
    gmb1 = gb[:, 224:256]
    gmb2 = gb[:, 256:288]
    gdb = gb[:, 288:312]
    fcb = gb[:, 312:336]

    gemW = gemW_ref[...]
    gWz = gWz_ref[...]
    gWr = gWr_ref[...]
    gWn = gWn_ref[...]
    exzr = jnp.concatenate([_dot(gemW, gWz), _dot(gemW, gWr)], axis=1)
    exn = _dot(gemW, gWn)
    bzr = jnp.concatenate([_dot(gemb, gWz) + gbz,
                           _dot(gemb, gWr) + gbr], axis=1)
    bn0 = _dot(gemb, gWn) + gbn
    Uzr = jnp.concatenate([gUz_ref[...], gUr_ref[...]], axis=1)
    Un = gUn_ref[...]

    h = jnp.zeros((CITY * B, H), _F32)
    for t in range(HIST):
        s = cxh_ref[:, t:t + 1]
        pzr = s * exzr + _dot(h, Uzr) + bzr
        z = jax.nn.sigmoid(pzr[:, :H])
        r = jax.nn.sigmoid(pzr[:, H:])
        nn = jnp.tanh(s * exn + _dot(r * h, Un) + bn0)
        h = (1.0 - z) * nn + z * h
    hg_ref[...] = h

    W1 = gmW1_ref[...]
    W1a = W1[:H, :]
    W1b = W1[H:2 * H, :]
    w1c = W1[2 * H:2 * H + 1, :]
    for e in range(CITY):
        si = cei_ref[0, e]
        di = cei_ref[1, e]
        gsrc_ref[e * B:(e + 1) * B, :] = hg_ref[pl.ds(si * B, B), :]
        gdst_ref[e * B:(e + 1) * B, :] = hg_ref[pl.ds(di * B, B), :]
        eawg_ref[e * B:(e + 1) * B, :] = jnp.broadcast_to(cea_ref[e, 0] * w1c, (B, G))
    m1 = jax.nn.relu(_dot(gsrc_ref[...], W1a) + _dot(gdst_ref[...], W1b)
                     + eawg_ref[...] + gmb1)
    m = _dot(m1, gmW2_ref[...]) + gmb2
    aggg_ref[...] = jnp.zeros((CITY * B, G), _F32)
    for e in range(CITY):
        di = cei_ref[1, e]
        aggg_ref[pl.ds(di * B, B), :] += m[e * B:(e + 1) * B, :]
    gd = gdW_ref[...]
    cu = _dot(h, gd[:H, :]) + _dot(aggg_ref[...], gd[H:, :]) + gdb

    maskH = _block_mask(H)
    maskG = _block_mask(G)
    maskP = _block_mask(PRED)
    maskE = _block_mask(AQI_EM)

    cemW = jnp.concatenate([cemW_ref[c] for c in range(CITY)], axis=0)
    cWzf = jnp.concatenate([cWz_ref[c] for c in range(CITY)], axis=0)
    cWrf = jnp.concatenate([cWr_ref[c] for c in range(CITY)], axis=0)
    cWnf = jnp.concatenate([cWn_ref[c] for c in range(CITY)], axis=0)

    emBD = maskE * _tile_lanes(cemW, CITY)
    ebBD = maskE * _tile_lanes(cemb_ref[...], CITY)
    exz_all = _dot(emBD, cWzf)
    exr_all = _dot(emBD, cWrf)
    exn_all = _dot(emBD, cWnf)
    bz_all = _dot(ebBD, cWzf) + cbz_ref[...]
    br_all = _dot(ebBD, cWrf) + cbr_ref[...]
    bn_all = _dot(ebBD, cWnf) + cbn_ref[...]

    XWz = maskH * _tile_lanes(exz_all, CITY)
    XWr = maskH * _tile_lanes(exr_all, CITY)
    XWzr = jnp.concatenate([XWz, XWr], axis=1).astype(_BF16)
    XWn = (maskH * _tile_lanes(exn_all, CITY)).astype(_BF16)
    bzr_row = jnp.concatenate([_to_row(bz_all, maskH), _to_row(br_all, maskH)], axis=1)
    bn_row = _to_row(bn_all, maskH)

    NG = 10
    SZ = CITY // NG
    GH = SZ * H
    UzrBD_ref[...] = jnp.zeros((GH, NG * 2 * GH), _BF16)
    UnBD_ref[...] = jnp.zeros((GH, NG * GH), _BF16)
    W1aBD_ref[...] = jnp.zeros((CH, CG), _BF16)
    W1bBD_ref[...] = jnp.zeros((CH, CG), _BF16)
    W2BD_ref[...] = jnp.zeros((CG, CG), _BF16)
    D1BD_ref[...] = jnp.zeros((CH, CP), _BF16)
    D2BD_ref[...] = jnp.zeros((CG, CP), _BF16)
    for c in range(CITY):
        hsl = slice(c * H, (c + 1) * H)
        gsl = slice(c * G, (c + 1) * G)
        psl = slice(c * PRED, (c + 1) * PRED)
        g = c // SZ
        i = c % SZ
        isl = slice(i * H, (i + 1) * H)
        zo = g * 2 * GH + i * H
        ro = g * 2 * GH + GH + i * H
        UzrBD_ref[isl, zo:zo + H] = cUz_ref[c].astype(_BF16)
        UzrBD_ref[isl, ro:ro + H] = cUr_ref[c].astype(_BF16)
        UnBD_ref[isl, g * GH + i * H:g * GH + (i + 1) * H] = cUn_ref[c].astype(_BF16)
        W1aBD_ref[hsl, gsl] = cmW1_ref[c, :H, :].astype(_BF16)
        W1bBD_ref[hsl, gsl] = cmW1_ref[c, H:2 * H, :].astype(_BF16)
        W2BD_ref[gsl, gsl] = cmW2_ref[c].astype(_BF16)
        D1BD_ref[hsl, psl] = cdW_ref[c, :H, :].astype(_BF16)
        D2BD_ref[gsl, psl] = cdW_ref[c, H:H + G, :].astype(_BF16)

    cm_acc = cmt_ref[0]
    for t in range(1, HIST):
        cm_acc = cm_acc + cmt_ref[t]
    chm = cm_acc * (1.0 / HIST)
    h0_all = jnp.concatenate(
        [_dot(chm[:, 4 * c:4 * c + 4], ch0W_ref[c]) for c in range(CITY)], axis=1
    )
    hv = jnp.concatenate([h0_all] * NSTA, axis=0)

    fcW = fcW_ref[...]
    Uzrg = [UzrBD_ref[:, g * 2 * GH:(g + 1) * 2 * GH] for g in range(NG)]
    Ung = [UnBD_ref[:, g * GH:(g + 1) * GH] for g in range(NG)]
    base = jnp.broadcast_to(_tile_lanes(fcb, CITY), (NB, CP))
    for t in range(HIST):
        s_t = xst_ref[t]
        FT_t = maskP * _tile_lanes(jnp.broadcast_to(fcW[t:t + 1, :], (CITY, PRED)), CITY)
        base = base + _dot(s_t, FT_t)
        s_tb = s_t.astype(_BF16)
        hvb = hv.astype(_BF16)
        px = _dot(s_tb, XWzr) + bzr_row
        pg = [_dot(hvb[:, g * GH:(g + 1) * GH], Uzrg[g]) for g in range(NG)]
        z = jax.nn.sigmoid(px[:, :CH]
                           + jnp.concatenate([p[:, :GH] for p in pg], axis=1))
        r = jax.nn.sigmoid(px[:, CH:]
                           + jnp.concatenate([p[:, GH:] for p in pg], axis=1))
        rh = (r * hv).astype(_BF16)
        nn = jnp.tanh(_dot(s_tb, XWn) + bn_row
                      + jnp.concatenate(
                          [_dot(rh[:, g * GH:(g + 1) * GH], Ung[g])
                           for g in range(NG)], axis=1))
        hv = (1.0 - z) * nn + z * hv
    hs_ref[...] = hv.astype(_BF16)

    w1c_all = cmW1_ref[:, 2 * H, :]
    w1c_row = _to_row(w1c_all, maskG)
    b1_row = _to_row(cmb1_ref[...], maskG)
    b2_row = _to_row(cmb2_ref[...], maskG)
    for e in range(NSTA):
        si = ei_ref[0, e]
        di = ei_ref[1, e]
        ssrc_ref[e * B:(e + 1) * B, :] = hs_ref[pl.ds(si * B, B), :]
        sdst_ref[e * B:(e + 1) * B, :] = hs_ref[pl.ds(di * B, B), :]
        eaws_ref[e * B:(e + 1) * B, :] = jnp.broadcast_to(sea_ref[e, 0] * w1c_row, (B, CG))
    mm1 = jax.nn.relu(_dot(ssrc_ref[...], W1aBD_ref[...])
                      + _dot(sdst_ref[...], W1bBD_ref[...])
                      + eaws_ref[...] + b1_row)
    mm = _dot(mm1.astype(_BF16), W2BD_ref[...]) + b2_row
    ags_ref[...] = jnp.zeros((NB, CG), _F32)
    for e in range(NSTA):
        di = ei_ref[1, e]
        ags_ref[pl.ds(di * B, B), :] += mm[e * B:(e + 1) * B, :]

    cdb_row = _to_row(cdb_ref[...], maskP)
    corr = _dot(hs_ref[...], D1BD_ref[...]) \
        + _dot(ags_ref[...].astype(_BF16), D2BD_ref[...]) + cdb_row

    Q = PRED * CITY * 2
    q_p = jax.lax.broadcasted_iota(jnp.int32, (Q, PRED), 0) // (CITY * 2)
    q_c = (jax.lax.broadcasted_iota(jnp.int32, (Q, PRED), 0) % (CITY * 2)) // 2
    q_j = jax.lax.broadcasted_iota(jnp.int32, (Q, PRED), 0) % 2
    o_p = jax.lax.broadcasted_iota(jnp.int32, (Q, PRED), 1)
    pmatch = q_p == o_p
    zeroQ = jnp.zeros((Q, PRED), _F32)
    m1_blocks = []
    m2_blocks = []
    for c in range(CITY):
        sel = pmatch & (q_c == c)
        m1_blocks.append(jnp.where(sel, jnp.where(q_j == 0, ccf_ref[c, 0, 0],
                                                  ccf_ref[c, 1, 0]), zeroQ))
        m2_blocks.append(jnp.where(sel, jnp.where(q_j == 0, ccf_ref[c, 2, 0],
                                                  ccf_ref[c, 3, 0]), zeroQ))
    M1 = jnp.concatenate(m1_blocks, axis=1)
    M2 = jnp.concatenate(m2_blocks, axis=1)
    cdd = cdd_ref[...]
    ct2 = _dot(cdd[:, :Q], M1) + _dot(cdd[:, Q:], M2)

    cur = jnp.concatenate([cu[c * B:(c + 1) * B, :] for c in range(CITY)], axis=1)
    add2 = ct2 + cur
    addb = jnp.concatenate([add2] * NSTA, axis=0)

    out_ref[...] = base + corr + addb


_STA_MARK = np.arange(STA, dtype=np.int32)


def kernel(x_hist, sta_misc, sta_dec_met, sta_dec_time, c_x_hist, c_misc,
           c_dec_met, c_dec_time, city_edge_index, city_edge_attr,
           edge_index, edge_attr, g_em_W, g_em_b, g_Wz, g_Uz, g_bz,
           g_Wr, g_Ur, g_br, g_Wn, g_Un, g_bn, g_msg_W1, g_msg_b1,
           g_msg_W2, g_msg_b2, g_dec_W, g_dec_b, c_em_W, c_em_b, c_h0_W,
           c_Wz, c_Uz, c_bz, c_Wr, c_Ur, c_br, c_Wn, c_Un, c_bn,
           c_msg_W1, c_msg_b1, c_msg_W2, c_msg_b2, c_dec_W, c_dec_b,
           c_cf_W, fc_W, fc_b):
    cxh = c_x_hist[..., 0].transpose(2, 0, 1).reshape(CITY * B, HIST)
    xst = (x_hist[..., 0].reshape(B, HIST, CITY, NSTA)
           .transpose(1, 3, 0, 2).reshape(HIST, NSTA * B, CITY))
    cmt = c_misc.transpose(1, 0, 2, 3).reshape(HIST, B, CITY * 4)
    cdd = jnp.concatenate([c_dec_met.reshape(B, PRED * CITY * 2),
                           c_dec_time.reshape(B, PRED * CITY * 2)], axis=1)
    gbias = jnp.concatenate([g_em_b, g_bz, g_br, g_bn, g_msg_b1,
                             g_msg_b2, g_dec_b, fc_b]).reshape(1, 336)

    vmem = pl.BlockSpec(memory_space=pltpu.VMEM)
    smem = pl.BlockSpec(memory_space=pltpu.SMEM)
    CH = CITY * RNN_H
    CG = CITY * GNN_H
    CP = CITY * PRED
    NB = NSTA * B

    out = pl.pallas_call(
        _fused_body,
        out_shape=jax.ShapeDtypeStruct((NB, CP), _F32),
        in_specs=[vmem] * 34 + [smem] * 5,
        out_specs=vmem,
        scratch_shapes=[
            pltpu.VMEM((CITY * B, RNN_H), _F32),
            pltpu.VMEM((CITY * B, RNN_H), _F32),
            pltpu.VMEM((CITY * B, RNN_H), _F32),
            pltpu.VMEM((CITY * B, GNN_H), _F32),
            pltpu.VMEM((CITY * B, GNN_H), _F32),
            pltpu.VMEM((NB, CH), _BF16),
            pltpu.VMEM((NB, CH), _BF16),
            pltpu.VMEM((NB, CH), _BF16),
            pltpu.VMEM((NB, CG), _F32),
            pltpu.VMEM((NB, CG), _F32),
            pltpu.VMEM((RNN_H, 2 * CH), _BF16),
            pltpu.VMEM((RNN_H, CH), _BF16),
            pltpu.VMEM((CH, CG), _BF16),
            pltpu.VMEM((CH, CG), _BF16),
            pltpu.VMEM((CG, CG), _BF16),
            pltpu.VMEM((CH, CP), _BF16),
            pltpu.VMEM((CG, CP), _BF16),
        ],
    )(
        cxh, xst, cmt, cdd, gbias,
        g_em_W, g_Wz, g_Wr, g_Wn, g_Uz, g_Ur, g_Un,
        g_msg_W1, g_msg_W2, g_dec_W,
        c_em_W, c_em_b, c_h0_W,
        c_Wz, c_Wr, c_Wn, c_Uz, c_Ur, c_Un, c_bz, c_br, c_bn,
        c_msg_W1, c_msg_b1, c_msg_W2, c_msg_b2,
        c_dec_W, c_dec_b, fc_W,
        city_edge_index, edge_index,
        city_edge_attr, edge_attr, c_cf_W,
    )

    out4 = (out.reshape(NSTA, B, CITY, PRED).transpose(1, 3, 2, 0)
            .reshape(B, PRED, STA, 1))
    return (out4, jnp.asarray(_STA_MARK))


# speedup vs baseline: 1.0466x; 1.0466x over previous
import numpy as np
import jax
import jax.numpy as jnp
from jax.experimental import pallas as pl
from jax.experimental.pallas import tpu as pltpu

B = 32
HIST = 8
PRED = 24
CITY = 10
NSTA = 12
STA = 120
AQI_EM = 32
RNN_H = 64
GNN_H = 32

_F32 = jnp.float32
_BF16 = jnp.bfloat16


def _dot(a, b):
    return jnp.dot(a, b, preferred_element_type=_F32)


def _block_mask(width):
    lane = jax.lax.broadcasted_iota(jnp.int32, (CITY, CITY * width), 1)
    sub = jax.lax.broadcasted_iota(jnp.int32, (CITY, CITY * width), 0)
    return jnp.where(lane // width == sub, 1.0, 0.0).astype(_F32)


def _tile_lanes(x, n):
    return jnp.concatenate([x] * n, axis=1)


def _to_row(per_city, mask):
    return jnp.sum(mask * _tile_lanes(per_city, CITY), axis=0, keepdims=True)


def _fused_body(
    cxh_ref,
    xst_ref,
    cmt_ref,
    cdd_ref,
    gbias_ref,
    gemW_ref,
    gWz_ref, gWr_ref, gWn_ref,
    gUz_ref, gUr_ref, gUn_ref,
    gmW1_ref,
    gmW2_ref,
    gdW_ref,
    cemW_ref,
    cemb_ref,
    ch0W_ref,
    cWz_ref, cWr_ref, cWn_ref,
    cUz_ref, cUr_ref, cUn_ref,
    cbz_ref, cbr_ref, cbn_ref,
    cmW1_ref,
    cmb1_ref,
    cmW2_ref,
    cmb2_ref,
    cdW_ref,
    cdb_ref,
    fcW_ref,
    cei_ref,
    ei_ref,
    cea_ref,
    sea_ref,
    ccf_ref,
    out_ref,
    hg_ref,
    gsrc_ref,
    gdst_ref,
    eawg_ref,
    aggg_ref,
    hs_ref,
    ssrc_ref,
    sdst_ref,
    eaws_ref,
    ags_ref,
    UzrBD_ref,
    UnBD_ref,
    W1aBD_ref,
    W1bBD_ref,
    W2BD_ref,
    D1BD_ref,
    D2BD_ref,
):
    H = RNN_H
    G = GNN_H
    CH = CITY * H
    CG = CITY * G
    CP = CITY * PRED
    NB = NSTA * B

    gb = gbias_ref[...]
    gemb = gb[:, 0:32]
    gbz = gb[:, 32:96]
    gbr = gb[:, 96:160]
    gbn = gb[:, 160:224]
    gmb1 = gb[:, 224:256]
    gmb2 = gb[:, 256:288]
    gdb = gb[:, 288:312]
    fcb = gb[:, 312:336]

    gemW = gemW_ref[...]
    gWz = gWz_ref[...]
    gWr = gWr_ref[...]
    gWn = gWn_ref[...]
    exzr = jnp.concatenate([_dot(gemW, gWz), _dot(gemW, gWr)], axis=1)
    exn = _dot(gemW, gWn)
    bzr = jnp.concatenate([_dot(gemb, gWz) + gbz,
                           _dot(gemb, gWr) + gbr], axis=1)
    bn0 = _dot(gemb, gWn) + gbn
    Uzr = jnp.concatenate([gUz_ref[...], gUr_ref[...]], axis=1)
    Un = gUn_ref[...]

    h = jnp.zeros((CITY * B, H), _F32)
    for t in range(HIST):
        s = cxh_ref[:, t:t + 1]
        pzr = s * exzr + _dot(h, Uzr) + bzr
        z = jax.nn.sigmoid(pzr[:, :H])
        r = jax.nn.sigmoid(pzr[:, H:])
        nn = jnp.tanh(s * exn + _dot(r * h, Un) + bn0)
        h = (1.0 - z) * nn + z * h
    hg_ref[...] = h

    W1 = gmW1_ref[...]
    W1a = W1[:H, :]
    W1b = W1[H:2 * H, :]
    w1c = W1[2 * H:2 * H + 1, :]
    for e in range(CITY):
        si = cei_ref[0, e]
        di = cei_ref[1, e]
        gsrc_ref[e * B:(e + 1) * B, :] = hg_ref[pl.ds(si * B, B), :]
        gdst_ref[e * B:(e + 1) * B, :] = hg_ref[pl.ds(di * B, B), :]
        eawg_ref[e * B:(e + 1) * B, :] = jnp.broadcast_to(cea_ref[e, 0] * w1c, (B, G))
    m1 = jax.nn.relu(_dot(gsrc_ref[...], W1a) + _dot(gdst_ref[...], W1b)
                     + eawg_ref[...] + gmb1)
    m = _dot(m1, gmW2_ref[...]) + gmb2
    aggg_ref[...] = jnp.zeros((CITY * B, G), _F32)
    for e in range(CITY):
        di = cei_ref[1, e]
        aggg_ref[pl.ds(di * B, B), :] += m[e * B:(e + 1) * B, :]
    gd = gdW_ref[...]
    cu = _dot(h, gd[:H, :]) + _dot(aggg_ref[...], gd[H:, :]) + gdb

    maskH = _block_mask(H)
    maskG = _block_mask(G)
    maskP = _block_mask(PRED)
    maskE = _block_mask(AQI_EM)

    cemW = jnp.concatenate([cemW_ref[c] for c in range(CITY)], axis=0)
    cWzf = jnp.concatenate([cWz_ref[c] for c in range(CITY)], axis=0)
    cWrf = jnp.concatenate([cWr_ref[c] for c in range(CITY)], axis=0)
    cWnf = jnp.concatenate([cWn_ref[c] for c in range(CITY)], axis=0)

    emBD = maskE * _tile_lanes(cemW, CITY)
    ebBD = maskE * _tile_lanes(cemb_ref[...], CITY)
    exz_all = _dot(emBD, cWzf)
    exr_all = _dot(emBD, cWrf)
    exn_all = _dot(emBD, cWnf)
    bz_all = _dot(ebBD, cWzf) + cbz_ref[...]
    br_all = _dot(ebBD, cWrf) + cbr_ref[...]
    bn_all = _dot(ebBD, cWnf) + cbn_ref[...]

    XWz = maskH * _tile_lanes(exz_all, CITY)
    XWr = maskH * _tile_lanes(exr_all, CITY)
    XWzr = jnp.concatenate([XWz, XWr], axis=1).astype(_BF16)
    XWn = (maskH * _tile_lanes(exn_all, CITY)).astype(_BF16)
    bzr_row = jnp.concatenate([_to_row(bz_all, maskH), _to_row(br_all, maskH)], axis=1)
    bn_row = _to_row(bn_all, maskH)

    NG = 5
    SZ = CITY // NG
    GH = SZ * H
    UzrBD_ref[...] = jnp.zeros((GH, NG * 2 * GH), _BF16)
    UnBD_ref[...] = jnp.zeros((GH, NG * GH), _BF16)
    W1aBD_ref[...] = jnp.zeros((CH, CG), _BF16)
    W1bBD_ref[...] = jnp.zeros((CH, CG), _BF16)
    W2BD_ref[...] = jnp.zeros((CG, CG), _BF16)
    D1BD_ref[...] = jnp.zeros((CH, CP), _BF16)
    D2BD_ref[...] = jnp.zeros((CG, CP), _BF16)
    for c in range(CITY):
        hsl = slice(c * H, (c + 1) * H)
        gsl = slice(c * G, (c + 1) * G)
        psl = slice(c * PRED, (c + 1) * PRED)
        g = c // SZ
        i = c % SZ
        isl = slice(i * H, (i + 1) * H)
        zo = g * 2 * GH + i * H
        ro = g * 2 * GH + GH + i * H
        UzrBD_ref[isl, zo:zo + H] = cUz_ref[c].astype(_BF16)
        UzrBD_ref[isl, ro:ro + H] = cUr_ref[c].astype(_BF16)
        UnBD_ref[isl, g * GH + i * H:g * GH + (i + 1) * H] = cUn_ref[c].astype(_BF16)
        W1aBD_ref[hsl, gsl] = cmW1_ref[c, :H, :].astype(_BF16)
        W1bBD_ref[hsl, gsl] = cmW1_ref[c, H:2 * H, :].astype(_BF16)
        W2BD_ref[gsl, gsl] = cmW2_ref[c].astype(_BF16)
        D1BD_ref[hsl, psl] = cdW_ref[c, :H, :].astype(_BF16)
        D2BD_ref[gsl, psl] = cdW_ref[c, H:H + G, :].astype(_BF16)

    cm_acc = cmt_ref[0]
    for t in range(1, HIST):
        cm_acc = cm_acc + cmt_ref[t]
    chm = cm_acc * (1.0 / HIST)
    h0_all = jnp.concatenate(
        [_dot(chm[:, 4 * c:4 * c + 4], ch0W_ref[c]) for c in range(CITY)], axis=1
    )
    hv = jnp.concatenate([h0_all] * NSTA, axis=0)

    fcW = fcW_ref[...]
    Uzrg = [UzrBD_ref[:, g * 2 * GH:(g + 1) * 2 * GH] for g in range(NG)]
    Ung = [UnBD_ref[:, g * GH:(g + 1) * GH] for g in range(NG)]
    base = jnp.broadcast_to(_tile_lanes(fcb, CITY), (NB, CP))
    for t in range(HIST):
        s_t = xst_ref[t]
        FT_t = maskP * _tile_lanes(jnp.broadcast_to(fcW[t:t + 1, :], (CITY, PRED)), CITY)
        base = base + _dot(s_t, FT_t)
        s_tb = s_t.astype(_BF16)
        hvb = hv.astype(_BF16)
        px = _dot(s_tb, XWzr) + bzr_row
        pg = [_dot(hvb[:, g * GH:(g + 1) * GH], Uzrg[g]) for g in range(NG)]
        z = jax.nn.sigmoid(px[:, :CH]
                           + jnp.concatenate([p[:, :GH] for p in pg], axis=1))
        r = jax.nn.sigmoid(px[:, CH:]
                           + jnp.concatenate([p[:, GH:] for p in pg], axis=1))
        rh = (r * hv).astype(_BF16)
        nn = jnp.tanh(_dot(s_tb, XWn) + bn_row
                      + jnp.concatenate(
                          [_dot(rh[:, g * GH:(g + 1) * GH], Ung[g])
                           for g in range(NG)], axis=1))
        hv = (1.0 - z) * nn + z * hv
    hs_ref[...] = hv.astype(_BF16)

    w1c_all = cmW1_ref[:, 2 * H, :]
    w1c_row = _to_row(w1c_all, maskG)
    b1_row = _to_row(cmb1_ref[...], maskG)
    b2_row = _to_row(cmb2_ref[...], maskG)
    for e in range(NSTA):
        si = ei_ref[0, e]
        di = ei_ref[1, e]
        ssrc_ref[e * B:(e + 1) * B, :] = hs_ref[pl.ds(si * B, B), :]
        sdst_ref[e * B:(e + 1) * B, :] = hs_ref[pl.ds(di * B, B), :]
        eaws_ref[e * B:(e + 1) * B, :] = jnp.broadcast_to(sea_ref[e, 0] * w1c_row, (B, CG))
    mm1 = jax.nn.relu(_dot(ssrc_ref[...], W1aBD_ref[...])
                      + _dot(sdst_ref[...], W1bBD_ref[...])
                      + eaws_ref[...] + b1_row)
    mm = _dot(mm1.astype(_BF16), W2BD_ref[...]) + b2_row
    ags_ref[...] = jnp.zeros((NB, CG), _F32)
    for e in range(NSTA):
        di = ei_ref[1, e]
        ags_ref[pl.ds(di * B, B), :] += mm[e * B:(e + 1) * B, :]

    cdb_row = _to_row(cdb_ref[...], maskP)
    corr = _dot(hs_ref[...], D1BD_ref[...]) \
        + _dot(ags_ref[...].astype(_BF16), D2BD_ref[...]) + cdb_row

    Q = PRED * CITY * 2
    q_p = jax.lax.broadcasted_iota(jnp.int32, (Q, PRED), 0) // (CITY * 2)
    q_c = (jax.lax.broadcasted_iota(jnp.int32, (Q, PRED), 0) % (CITY * 2)) // 2
    q_j = jax.lax.broadcasted_iota(jnp.int32, (Q, PRED), 0) % 2
    o_p = jax.lax.broadcasted_iota(jnp.int32, (Q, PRED), 1)
    pmatch = q_p == o_p
    zeroQ = jnp.zeros((Q, PRED), _F32)
    m1_blocks = []
    m2_blocks = []
    for c in range(CITY):
        sel = pmatch & (q_c == c)
        m1_blocks.append(jnp.where(sel, jnp.where(q_j == 0, ccf_ref[c, 0, 0],
                                                  ccf_ref[c, 1, 0]), zeroQ))
        m2_blocks.append(jnp.where(sel, jnp.where(q_j == 0, ccf_ref[c, 2, 0],
                                                  ccf_ref[c, 3, 0]), zeroQ))
    M1 = jnp.concatenate(m1_blocks, axis=1)
    M2 = jnp.concatenate(m2_blocks, axis=1)
    cdd = cdd_ref[...]
    ct2 = _dot(cdd[:, :Q], M1) + _dot(cdd[:, Q:], M2)

    cur = jnp.concatenate([cu[c * B:(c + 1) * B, :] for c in range(CITY)], axis=1)
    add2 = ct2 + cur
    addb = jnp.concatenate([add2] * NSTA, axis=0)

    out_ref[...] = base + corr + addb


_STA_MARK = np.arange(STA, dtype=np.int32)


def kernel(x_hist, sta_misc, sta_dec_met, sta_dec_time, c_x_hist, c_misc,
           c_dec_met, c_dec_time, city_edge_index, city_edge_attr,
           edge_index, edge_attr, g_em_W, g_em_b, g_Wz, g_Uz, g_bz,
           g_Wr, g_Ur, g_br, g_Wn, g_Un, g_bn, g_msg_W1, g_msg_b1,
           g_msg_W2, g_msg_b2, g_dec_W, g_dec_b, c_em_W, c_em_b, c_h0_W,
           c_Wz, c_Uz, c_bz, c_Wr, c_Ur, c_br, c_Wn, c_Un, c_bn,
           c_msg_W1, c_msg_b1, c_msg_W2, c_msg_b2, c_dec_W, c_dec_b,
           c_cf_W, fc_W, fc_b):
    cxh = c_x_hist[..., 0].transpose(2, 0, 1).reshape(CITY * B, HIST)
    xst = (x_hist[..., 0].reshape(B, HIST, CITY, NSTA)
           .transpose(1, 3, 0, 2).reshape(HIST, NSTA * B, CITY))
    cmt = c_misc.transpose(1, 0, 2, 3).reshape(HIST, B, CITY * 4)
    cdd = jnp.concatenate([c_dec_met.reshape(B, PRED * CITY * 2),
                           c_dec_time.reshape(B, PRED * CITY * 2)], axis=1)
    gbias = jnp.concatenate([g_em_b, g_bz, g_br, g_bn, g_msg_b1,
                             g_msg_b2, g_dec_b, fc_b]).reshape(1, 336)

    vmem = pl.BlockSpec(memory_space=pltpu.VMEM)
    smem = pl.BlockSpec(memory_space=pltpu.SMEM)
    CH = CITY * RNN_H
    CG = CITY * GNN_H
    CP = CITY * PRED
    NB = NSTA * B

    out = pl.pallas_call(
        _fused_body,
        out_shape=jax.ShapeDtypeStruct((NB, CP), _F32),
        in_specs=[vmem] * 34 + [smem] * 5,
        out_specs=vmem,
        scratch_shapes=[
            pltpu.VMEM((CITY * B, RNN_H), _F32),
            pltpu.VMEM((CITY * B, RNN_H), _F32),
            pltpu.VMEM((CITY * B, RNN_H), _F32),
            pltpu.VMEM((CITY * B, GNN_H), _F32),
            pltpu.VMEM((CITY * B, GNN_H), _F32),
            pltpu.VMEM((NB, CH), _BF16),
            pltpu.VMEM((NB, CH), _BF16),
            pltpu.VMEM((NB, CH), _BF16),
            pltpu.VMEM((NB, CG), _F32),
            pltpu.VMEM((NB, CG), _F32),
            pltpu.VMEM((2 * RNN_H, 2 * CH), _BF16),
            pltpu.VMEM((2 * RNN_H, CH), _BF16),
            pltpu.VMEM((CH, CG), _BF16),
            pltpu.VMEM((CH, CG), _BF16),
            pltpu.VMEM((CG, CG), _BF16),
            pltpu.VMEM((CH, CP), _BF16),
            pltpu.VMEM((CG, CP), _BF16),
        ],
    )(
        cxh, xst, cmt, cdd, gbias,
        g_em_W, g_Wz, g_Wr, g_Wn, g_Uz, g_Ur, g_Un,
        g_msg_W1, g_msg_W2, g_dec_W,
        c_em_W, c_em_b, c_h0_W,
        c_Wz, c_Wr, c_Wn, c_Uz, c_Ur, c_Un, c_bz, c_br, c_bn,
        c_msg_W1, c_msg_b1, c_msg_W2, c_msg_b2,
        c_dec_W, c_dec_b, fc_W,
        city_edge_index, edge_index,
        city_edge_attr, edge_attr, c_cf_W,
    )

    out4 = (out.reshape(NSTA, B, CITY, PRED).transpose(1, 3, 2, 0)
            .reshape(B, PRED, STA, 1))
    return (out4, jnp.asarray(_STA_MARK))
